# Initial kernel scaffold; baseline (speedup 1.0000x reference)
import jax
import jax.numpy as jnp
from jax.experimental import pallas as pl


def kernel(x, edge_index, params):
    raise NotImplementedError("write your pallas kernel here")


# speedup vs baseline: 12.1514x; 12.1514x over previous
import functools

import jax
import jax.numpy as jnp
from jax import lax
from jax.experimental import pallas as pl
from jax.experimental.pallas import tpu as pltpu
from jax.experimental.pallas import tpu_sc as plsc

N = 10000
E = 320000
D = 128
H = 8
DH = 16

NC = 2
NS = 16
DHALF = D // NC
HHALF = H // NC
EPT = E // NS
EB = 80
NBATCH = EPT // EB
RPT = 624

_SCALE = 1.0 / (DH ** 0.5)


def _edge_body(q_hbm, kv_hbm, src_hbm, dst_hbm, z64_hbm, z8_hbm,
               aggu_out, s_out,
               src_v, dst_v, kvrows, qrows, evbuf, ebuf,
               aggu_sp, s_sp, semk, semq):
    c = lax.axis_index("c")
    s = lax.axis_index("s")

    row0 = s * RPT
    pltpu.sync_copy(z64_hbm.at[pl.ds(0, RPT)], aggu_sp.at[pl.ds(row0, RPT)])
    pltpu.sync_copy(z8_hbm.at[pl.ds(0, RPT)], s_sp.at[pl.ds(row0, RPT)])
    pltpu.sync_copy(z8_hbm.at[pl.ds(0, EB)], ebuf)

    @pl.when(s == NS - 1)
    def _tail_zero():
        pltpu.sync_copy(z64_hbm.at[pl.ds(0, N - NS * RPT)],
                        aggu_sp.at[pl.ds(NS * RPT, N - NS * RPT)])
        pltpu.sync_copy(z8_hbm.at[pl.ds(0, N - NS * RPT)],
                        s_sp.at[pl.ds(NS * RPT, N - NS * RPT)])
    plsc.subcore_barrier()

    ebase = s * EPT
    lanes = lax.iota(jnp.int32, 16)

    def _batch(b, carry):
        off = ebase + b * EB
        pltpu.sync_copy(src_hbm.at[pl.ds(off, EB)], src_v)
        pltpu.sync_copy(dst_hbm.at[pl.ds(off, EB)], dst_v)
        cpq = pltpu.async_copy(q_hbm.at[dst_v], qrows, semq)
        cpk = pltpu.async_copy(kv_hbm.at[c].at[src_v], kvrows, semk)
        cpq.wait()
        cpk.wait()

        def _group(g, gcarry):
            eids = g * 16 + lanes
            for h in range(HHALF):
                acc = jnp.zeros((16,), jnp.float32)
                for f in range(DH):
                    kcol = jnp.full((16,), h * DH + f, jnp.int32)
                    qcol = kcol + c * DHALF
                    kt = plsc.load_gather(kvrows, [eids, kcol])
                    qt = plsc.load_gather(qrows, [eids, qcol])
                    acc = acc + kt * qt
                ev = jnp.exp(acc * _SCALE)
                ecol = jnp.full((16,), h, jnp.int32) + c * HHALF
                plsc.store_scatter(ebuf, [eids, ecol], ev)
                for f in range(DH):
                    col = jnp.full((16,), h * DH + f, jnp.int32)
                    vt = plsc.load_gather(kvrows, [eids, col + DHALF])
                    plsc.store_scatter(evbuf, [eids, col], vt * ev)
            return gcarry
        lax.fori_loop(0, EB // 16, _group, 0)
        pltpu.sync_copy(evbuf, aggu_sp.at[dst_v], add=True)
        pltpu.sync_copy(ebuf, s_sp.at[dst_v], add=True)
        return carry
    lax.fori_loop(0, NBATCH, _batch, 0)

    plsc.subcore_barrier()
    pltpu.sync_copy(aggu_sp.at[pl.ds(row0, RPT)],
                    aggu_out.at[c, pl.ds(row0, RPT)])
    pltpu.sync_copy(s_sp.at[pl.ds(row0, RPT)],
                    s_out.at[c, pl.ds(row0, RPT)])

    @pl.when(s == NS - 1)
    def _tail_out():
        pltpu.sync_copy(aggu_sp.at[pl.ds(NS * RPT, N - NS * RPT)],
                        aggu_out.at[c, pl.ds(NS * RPT, N - NS * RPT)])
        pltpu.sync_copy(s_sp.at[pl.ds(NS * RPT, N - NS * RPT)],
                        s_out.at[c, pl.ds(NS * RPT, N - NS * RPT)])


_edge_kernel = functools.partial(
    pl.kernel,
    out_type=[
        jax.ShapeDtypeStruct((NC, N, DHALF), jnp.float32),
        jax.ShapeDtypeStruct((NC, N, H), jnp.float32),
    ],
    mesh=plsc.VectorSubcoreMesh(core_axis_name="c", subcore_axis_name="s"),
    compiler_params=pltpu.CompilerParams(
        needs_layout_passes=False, use_tc_tiling_on_sc=False),
    scratch_types=[
        pltpu.VMEM((EB,), jnp.int32),
        pltpu.VMEM((EB,), jnp.int32),
        pltpu.VMEM((EB, D), jnp.float32),
        pltpu.VMEM((EB, D), jnp.float32),
        pltpu.VMEM((EB, DHALF), jnp.float32),
        pltpu.VMEM((EB, H), jnp.float32),
        pltpu.VMEM_SHARED((N, DHALF), jnp.float32),
        pltpu.VMEM_SHARED((N, H), jnp.float32),
        pltpu.SemaphoreType.DMA,
        pltpu.SemaphoreType.DMA,
    ],
)(_edge_body)


BN = 400


def _ln(h, g, b):
    mu = jnp.mean(h, axis=-1, keepdims=True)
    var = jnp.mean((h - mu) ** 2, axis=-1, keepdims=True)
    return (h - mu) * lax.rsqrt(var + 1e-5) * g + b


def _dot(a, b):
    return jnp.dot(a, b, preferred_element_type=jnp.float32)


def _post_attn(ap, sp, h_in, wo, bo, g1, b1, w1, bf1, w2, bf2, g2, b2):
    aggu = jnp.concatenate([ap[0], ap[1]], axis=-1)
    s8 = sp[0] + sp[1]
    rows = lax.broadcasted_iota(jnp.int32, (H, D), 0)
    cols = lax.broadcasted_iota(jnp.int32, (H, D), 1)
    expand = (cols // DH == rows).astype(jnp.float32)
    denom = _dot(s8, expand) + 1e-9
    agg = aggu / denom
    attn = _dot(agg, wo) + bo + h_in
    h1 = _ln(attn, g1, b1)
    ff = _dot(jax.nn.relu(_dot(h1, w1) + bf1), w2) + bf2
    return _ln(h1 + ff, g2, b2)


def _split_qkv(h2, wq, bq, wk, bk, wv, bv, q_out, kv_out):
    q_out[...] = _dot(h2, wq) + bq
    k = _dot(h2, wk) + bk
    v = _dot(h2, wv) + bv
    kv_out[0] = jnp.concatenate([k[:, :DHALF], v[:, :DHALF]], axis=1)
    kv_out[1] = jnp.concatenate([k[:, DHALF:], v[:, DHALF:]], axis=1)


def _qkv_body(x_ref, wq, bq, wk, bk, wv, bv, q_out, kv_out):
    _split_qkv(x_ref[...], wq[...], bq[...], wk[...], bk[...], wv[...], bv[...],
               q_out, kv_out)


def _mid_body(h_ref, ap_ref, sp_ref,
              wo, bo, g1, b1, w1, bf1, w2, bf2, g2, b2,
              wq, bq, wk, bk, wv, bv,
              h_out, q_out, kv_out):
    h2 = _post_attn(ap_ref, sp_ref, h_ref[...],
                    wo[...], bo[...], g1[...], b1[...], w1[...], bf1[...],
                    w2[...], bf2[...], g2[...], b2[...])
    h_out[...] = h2
    _split_qkv(h2, wq[...], bq[...], wk[...], bk[...], wv[...], bv[...],
               q_out, kv_out)


def _final_body(h_ref, ap_ref, sp_ref,
                wo, bo, g1, b1, w1, bf1, w2, bf2, g2, b2,
                r0w, r0b, r1w, r1b, r2w, r2b,
                out_ref):
    h2 = _post_attn(ap_ref, sp_ref, h_ref[...],
                    wo[...], bo[...], g1[...], b1[...], w1[...], bf1[...],
                    w2[...], bf2[...], g2[...], b2[...])
    hc = jnp.concatenate([h_ref[...], h2], axis=1)
    r = jax.nn.relu(_dot(hc, r0w[...]) + r0b[...])
    r = jax.nn.relu(_dot(r, r1w[...]) + r1b[...])
    out_ref[...] = _dot(r, r2w[...]) + r2b[...]


def _rowspec(cols):
    return pl.BlockSpec((BN, cols), lambda i: (i, 0))


def _fullspec(shape):
    nd = len(shape)
    return pl.BlockSpec(shape, lambda i, _nd=nd: (0,) * _nd)


def _partspec(cols):
    return pl.BlockSpec((NC, BN, cols), lambda i: (0, i, 0))


def _w(p, name):
    arr = p[name]
    if arr.ndim == 1:
        arr = arr.reshape(1, -1)
    return arr


def kernel(x, edge_index, params):
    src = edge_index[0]
    dst = edge_index[1]
    z64 = jnp.zeros((N, DHALF), jnp.float32)
    z8 = jnp.zeros((N, H), jnp.float32)
    p0, p1 = params["layers"]
    (r0w, r0b), (r1w, r1b), (r2w, r2b) = params["readout"]

    grid = (N // BN,)

    qkv_call = pl.pallas_call(
        _qkv_body,
        grid=grid,
        in_specs=[_rowspec(D)] + [_fullspec(s) for s in
                  [(D, D), (1, D), (D, D), (1, D), (D, D), (1, D)]],
        out_specs=[_rowspec(D), _partspec(D)],
        out_shape=[jax.ShapeDtypeStruct((N, D), jnp.float32),
                   jax.ShapeDtypeStruct((NC, N, D), jnp.float32)],
    )

    q0, kv0 = qkv_call(
        x, p0["Wq"], _w(p0, "bq"), p0["Wk"], _w(p0, "bk"),
        p0["Wv"], _w(p0, "bv"))

    ap0, sp0 = _edge_kernel(q0, kv0, src, dst, z64, z8)

    post_w_shapes = [(D, D), (1, D), (1, D), (1, D), (D, 2 * D), (1, 2 * D),
                     (2 * D, D), (1, D), (1, D), (1, D)]
    qkv_w_shapes = [(D, D), (1, D)] * 3

    mid_call = pl.pallas_call(
        _mid_body,
        grid=grid,
        in_specs=[_rowspec(D), _partspec(DHALF), _partspec(H)]
                 + [_fullspec(s) for s in post_w_shapes + qkv_w_shapes],
        out_specs=[_rowspec(D), _rowspec(D), _partspec(D)],
        out_shape=[jax.ShapeDtypeStruct((N, D), jnp.float32),
                   jax.ShapeDtypeStruct((N, D), jnp.float32),
                   jax.ShapeDtypeStruct((NC, N, D), jnp.float32)],
    )

    h1, q1, kv1 = mid_call(
        x, ap0, sp0,
        p0["Wo"], _w(p0, "bo"), _w(p0, "ln1_g"), _w(p0, "ln1_b"),
        p0["W1"], _w(p0, "b1"), p0["W2"], _w(p0, "b2"),
        _w(p0, "ln2_g"), _w(p0, "ln2_b"),
        p1["Wq"], _w(p1, "bq"), p1["Wk"], _w(p1, "bk"),
        p1["Wv"], _w(p1, "bv"))

    ap1, sp1 = _edge_kernel(q1, kv1, src, dst, z64, z8)

    readout_shapes = [(2 * D, D), (1, D), (D, D // 2), (1, D // 2),
                      (D // 2, 10), (1, 10)]

    final_call = pl.pallas_call(
        _final_body,
        grid=grid,
        in_specs=[_rowspec(D), _partspec(DHALF), _partspec(H)]
                 + [_fullspec(s) for s in post_w_shapes + readout_shapes],
        out_specs=_rowspec(10),
        out_shape=jax.ShapeDtypeStruct((N, 10), jnp.float32),
    )

    out = final_call(
        h1, ap1, sp1,
        p1["Wo"], _w(p1, "bo"), _w(p1, "ln1_g"), _w(p1, "ln1_b"),
        p1["W1"], _w(p1, "b1"), p1["W2"], _w(p1, "b2"),
        _w(p1, "ln2_g"), _w(p1, "ln2_b"),
        r0w, r0b.reshape(1, -1), r1w, r1b.reshape(1, -1),
        r2w, r2b.reshape(1, -1))

    return out


# speedup vs baseline: 14.6488x; 1.2055x over previous
import functools

import jax
import jax.numpy as jnp
from jax import lax
from jax.experimental import pallas as pl
from jax.experimental.pallas import tpu as pltpu
from jax.experimental.pallas import tpu_sc as plsc

N = 10000
E = 320000
D = 128
H = 8
DH = 16

NC = 2
NS = 16
DHALF = D // NC
HHALF = H // NC
EPT = E // NS
EB = 80
NBATCH = EPT // EB
CHUNK = 50
NCHUNK = NBATCH // CHUNK
RPT = 624

_SCALE = 1.0 / (DH ** 0.5)


def _edge_body(q_hbm, kv_hbm, packed_hbm, z64_hbm, z8_hbm,
               aggu_out, s_out,
               packed_all, src_all, dst_all,
               kv0, q0, ev0, eb0, kv1, q1, ev1, eb1,
               aggu_sp, s_sp,
               gq0, gk0, gq1, gk1, sa0, ss0, sa1, ss1):
    c = lax.axis_index("c")
    s = lax.axis_index("s")

    row0 = s * RPT
    pltpu.sync_copy(z64_hbm.at[pl.ds(0, RPT)], aggu_sp.at[pl.ds(row0, RPT)])
    pltpu.sync_copy(z8_hbm.at[pl.ds(0, RPT)], s_sp.at[pl.ds(row0, RPT)])
    pltpu.sync_copy(z8_hbm.at[pl.ds(0, EB)], eb0)
    pltpu.sync_copy(z8_hbm.at[pl.ds(0, EB)], eb1)

    @pl.when(s == NS - 1)
    def _tail_zero():
        pltpu.sync_copy(z64_hbm.at[pl.ds(0, N - NS * RPT)],
                        aggu_sp.at[pl.ds(NS * RPT, N - NS * RPT)])
        pltpu.sync_copy(z8_hbm.at[pl.ds(0, N - NS * RPT)],
                        s_sp.at[pl.ds(NS * RPT, N - NS * RPT)])

    plsc.subcore_barrier()

    lanes = lax.iota(jnp.int32, 16)

    def _compute(kvrows, qrows, evbuf, ebuf):
        def _group(g, gcarry):
            eids = g * 16 + lanes
            for h in range(HHALF):
                acc = jnp.zeros((16,), jnp.float32)
                for f in range(DH):
                    kcol = jnp.full((16,), h * DH + f, jnp.int32)
                    qcol = kcol + c * DHALF
                    kt = plsc.load_gather(kvrows, [eids, kcol])
                    qt = plsc.load_gather(qrows, [eids, qcol])
                    acc = acc + kt * qt
                ev = jnp.exp(acc * _SCALE)
                ecol = jnp.full((16,), h, jnp.int32) + c * HHALF
                plsc.store_scatter(ebuf, [eids, ecol], ev)
                for f in range(DH):
                    col = jnp.full((16,), h * DH + f, jnp.int32)
                    vt = plsc.load_gather(kvrows, [eids, col + DHALF])
                    plsc.store_scatter(evbuf, [eids, col], vt * ev)
            return gcarry
        lax.fori_loop(0, EB // 16, _group, 0)

    def _issue_gathers(b, kvrows, qrows, gq, gk):
        pltpu.async_copy(q_hbm.at[dst_all.at[b]], qrows, gq)
        pltpu.async_copy(kv_hbm.at[c].at[src_all.at[b]], kvrows, gk)

    def _wait_gathers(b, kvrows, qrows, gq, gk):
        pltpu.make_async_copy(q_hbm.at[dst_all.at[b]], qrows, gq).wait()
        pltpu.make_async_copy(kv_hbm.at[c].at[src_all.at[b]], kvrows, gk).wait()

    def _wait_scatters(b, evbuf, ebuf, sa, ss):
        pltpu.make_async_copy(evbuf, aggu_sp.at[dst_all.at[b]], sa).wait()
        pltpu.make_async_copy(ebuf, s_sp.at[dst_all.at[b]], ss).wait()

    def _chunk(ci, ccarry):
        pltpu.sync_copy(packed_hbm.at[s, ci], packed_all)

        def _unpack(b, carry):
            for j in range(EB // 16):
                p = packed_all[b, pl.ds(j * 16, 16)]
                src_all[b, pl.ds(j * 16, 16)] = lax.shift_right_logical(p, 14)
                dst_all[b, pl.ds(j * 16, 16)] = lax.bitwise_and(p, 16383)
            return carry
        lax.fori_loop(0, CHUNK, _unpack, 0)

        _issue_gathers(0, kv0, q0, gq0, gk0)

        def _iter(i, carry):
            b0 = 2 * i
            b1 = b0 + 1
            _issue_gathers(b1, kv1, q1, gq1, gk1)
            _wait_gathers(b0, kv0, q0, gq0, gk0)

            @pl.when(i > 0)
            def _():
                _wait_scatters(b0, ev0, eb0, sa0, ss0)
            _compute(kv0, q0, ev0, eb0)
            pltpu.async_copy(ev0, aggu_sp.at[dst_all.at[b0]], sa0, add=True)
            pltpu.async_copy(eb0, s_sp.at[dst_all.at[b0]], ss0, add=True)

            @pl.when(i < CHUNK // 2 - 1)
            def _():
                _issue_gathers(b0 + 2, kv0, q0, gq0, gk0)
            _wait_gathers(b1, kv1, q1, gq1, gk1)

            @pl.when(i > 0)
            def _():
                _wait_scatters(b1, ev1, eb1, sa1, ss1)
            _compute(kv1, q1, ev1, eb1)
            pltpu.async_copy(ev1, aggu_sp.at[dst_all.at[b1]], sa1, add=True)
            pltpu.async_copy(eb1, s_sp.at[dst_all.at[b1]], ss1, add=True)
            return carry
        lax.fori_loop(0, CHUNK // 2, _iter, 0)
        _wait_scatters(0, ev0, eb0, sa0, ss0)
        _wait_scatters(0, ev1, eb1, sa1, ss1)
        return ccarry
    lax.fori_loop(0, NCHUNK, _chunk, 0)

    plsc.subcore_barrier()
    pltpu.sync_copy(aggu_sp.at[pl.ds(row0, RPT)],
                    aggu_out.at[c, pl.ds(row0, RPT)])
    pltpu.sync_copy(s_sp.at[pl.ds(row0, RPT)],
                    s_out.at[c, pl.ds(row0, RPT)])

    @pl.when(s == NS - 1)
    def _tail_out():
        pltpu.sync_copy(aggu_sp.at[pl.ds(NS * RPT, N - NS * RPT)],
                        aggu_out.at[c, pl.ds(NS * RPT, N - NS * RPT)])
        pltpu.sync_copy(s_sp.at[pl.ds(NS * RPT, N - NS * RPT)],
                        s_out.at[c, pl.ds(NS * RPT, N - NS * RPT)])


_edge_kernel = functools.partial(
    pl.kernel,
    out_type=[
        jax.ShapeDtypeStruct((NC, N, DHALF), jnp.float32),
        jax.ShapeDtypeStruct((NC, N, H), jnp.float32),
    ],
    mesh=plsc.VectorSubcoreMesh(core_axis_name="c", subcore_axis_name="s"),
    compiler_params=pltpu.CompilerParams(
        needs_layout_passes=False, use_tc_tiling_on_sc=False),
    scratch_types=(
        [
            pltpu.VMEM((CHUNK, EB), jnp.int32),
            pltpu.VMEM((CHUNK, EB), jnp.int32),
            pltpu.VMEM((CHUNK, EB), jnp.int32),
        ]
        + [
            pltpu.VMEM((EB, D), jnp.float32),
            pltpu.VMEM((EB, D), jnp.float32),
            pltpu.VMEM((EB, DHALF), jnp.float32),
            pltpu.VMEM((EB, H), jnp.float32),
        ] * 2
        + [
            pltpu.VMEM_SHARED((N, DHALF), jnp.float32),
            pltpu.VMEM_SHARED((N, H), jnp.float32),
        ]
        + [pltpu.SemaphoreType.DMA] * 8
    ),
)(_edge_body)


BN = 400


def _ln(h, g, b):
    mu = jnp.mean(h, axis=-1, keepdims=True)
    var = jnp.mean((h - mu) ** 2, axis=-1, keepdims=True)
    return (h - mu) * lax.rsqrt(var + 1e-5) * g + b


def _dot(a, b):
    return jnp.dot(a, b, preferred_element_type=jnp.float32)


def _post_attn(ap, sp, h_in, wo, bo, g1, b1, w1, bf1, w2, bf2, g2, b2):
    aggu = jnp.concatenate([ap[0], ap[1]], axis=-1)
    s8 = sp[0] + sp[1]
    rows = lax.broadcasted_iota(jnp.int32, (H, D), 0)
    cols = lax.broadcasted_iota(jnp.int32, (H, D), 1)
    expand = (cols // DH == rows).astype(jnp.float32)
    denom = _dot(s8, expand) + 1e-9
    agg = aggu / denom
    attn = _dot(agg, wo) + bo + h_in
    h1 = _ln(attn, g1, b1)
    ff = _dot(jax.nn.relu(_dot(h1, w1) + bf1), w2) + bf2
    return _ln(h1 + ff, g2, b2)


def _split_qkv(h2, wq, bq, wk, bk, wv, bv, q_out, kv_out):
    q_out[...] = _dot(h2, wq) + bq
    k = _dot(h2, wk) + bk
    v = _dot(h2, wv) + bv
    kv_out[0] = jnp.concatenate([k[:, :DHALF], v[:, :DHALF]], axis=1)
    kv_out[1] = jnp.concatenate([k[:, DHALF:], v[:, DHALF:]], axis=1)


def _qkv_body(x_ref, wq, bq, wk, bk, wv, bv, q_out, kv_out):
    _split_qkv(x_ref[...], wq[...], bq[...], wk[...], bk[...], wv[...], bv[...],
               q_out, kv_out)


def _mid_body(h_ref, ap_ref, sp_ref,
              wo, bo, g1, b1, w1, bf1, w2, bf2, g2, b2,
              wq, bq, wk, bk, wv, bv,
              h_out, q_out, kv_out):
    h2 = _post_attn(ap_ref, sp_ref, h_ref[...],
                    wo[...], bo[...], g1[...], b1[...], w1[...], bf1[...],
                    w2[...], bf2[...], g2[...], b2[...])
    h_out[...] = h2
    _split_qkv(h2, wq[...], bq[...], wk[...], bk[...], wv[...], bv[...],
               q_out, kv_out)


def _final_body(h_ref, ap_ref, sp_ref,
                wo, bo, g1, b1, w1, bf1, w2, bf2, g2, b2,
                r0w, r0b, r1w, r1b, r2w, r2b,
                out_ref):
    h2 = _post_attn(ap_ref, sp_ref, h_ref[...],
                    wo[...], bo[...], g1[...], b1[...], w1[...], bf1[...],
                    w2[...], bf2[...], g2[...], b2[...])
    hc = jnp.concatenate([h_ref[...], h2], axis=1)
    r = jax.nn.relu(_dot(hc, r0w[...]) + r0b[...])
    r = jax.nn.relu(_dot(r, r1w[...]) + r1b[...])
    out_ref[...] = _dot(r, r2w[...]) + r2b[...]


def _rowspec(cols):
    return pl.BlockSpec((BN, cols), lambda i: (i, 0))


def _fullspec(shape):
    nd = len(shape)
    return pl.BlockSpec(shape, lambda i, _nd=nd: (0,) * _nd)


def _partspec(cols):
    return pl.BlockSpec((NC, BN, cols), lambda i: (0, i, 0))


def _w(p, name):
    arr = p[name]
    if arr.ndim == 1:
        arr = arr.reshape(1, -1)
    return arr


def kernel(x, edge_index, params):
    packed = (jnp.left_shift(edge_index[0], 14) | edge_index[1]).reshape(
        NS, NCHUNK, CHUNK, EB)
    z64 = jnp.zeros((N, DHALF), jnp.float32)
    z8 = jnp.zeros((N, H), jnp.float32)
    p0, p1 = params["layers"]
    (r0w, r0b), (r1w, r1b), (r2w, r2b) = params["readout"]

    grid = (N // BN,)

    qkv_call = pl.pallas_call(
        _qkv_body,
        grid=grid,
        in_specs=[_rowspec(D)] + [_fullspec(s) for s in
                  [(D, D), (1, D), (D, D), (1, D), (D, D), (1, D)]],
        out_specs=[_rowspec(D), _partspec(D)],
        out_shape=[jax.ShapeDtypeStruct((N, D), jnp.float32),
                   jax.ShapeDtypeStruct((NC, N, D), jnp.float32)],
    )

    q0, kv0 = qkv_call(
        x, p0["Wq"], _w(p0, "bq"), p0["Wk"], _w(p0, "bk"),
        p0["Wv"], _w(p0, "bv"))

    ap0, sp0 = _edge_kernel(q0, kv0, packed, z64, z8)

    post_w_shapes = [(D, D), (1, D), (1, D), (1, D), (D, 2 * D), (1, 2 * D),
                     (2 * D, D), (1, D), (1, D), (1, D)]
    qkv_w_shapes = [(D, D), (1, D)] * 3

    mid_call = pl.pallas_call(
        _mid_body,
        grid=grid,
        in_specs=[_rowspec(D), _partspec(DHALF), _partspec(H)]
                 + [_fullspec(s) for s in post_w_shapes + qkv_w_shapes],
        out_specs=[_rowspec(D), _rowspec(D), _partspec(D)],
        out_shape=[jax.ShapeDtypeStruct((N, D), jnp.float32),
                   jax.ShapeDtypeStruct((N, D), jnp.float32),
                   jax.ShapeDtypeStruct((NC, N, D), jnp.float32)],
    )

    h1, q1, kv1 = mid_call(
        x, ap0, sp0,
        p0["Wo"], _w(p0, "bo"), _w(p0, "ln1_g"), _w(p0, "ln1_b"),
        p0["W1"], _w(p0, "b1"), p0["W2"], _w(p0, "b2"),
        _w(p0, "ln2_g"), _w(p0, "ln2_b"),
        p1["Wq"], _w(p1, "bq"), p1["Wk"], _w(p1, "bk"),
        p1["Wv"], _w(p1, "bv"))

    ap1, sp1 = _edge_kernel(q1, kv1, packed, z64, z8)

    readout_shapes = [(2 * D, D), (1, D), (D, D // 2), (1, D // 2),
                      (D // 2, 10), (1, 10)]

    final_call = pl.pallas_call(
        _final_body,
        grid=grid,
        in_specs=[_rowspec(D), _partspec(DHALF), _partspec(H)]
                 + [_fullspec(s) for s in post_w_shapes + readout_shapes],
        out_specs=_rowspec(10),
        out_shape=jax.ShapeDtypeStruct((N, 10), jnp.float32),
    )

    out = final_call(
        h1, ap1, sp1,
        p1["Wo"], _w(p1, "bo"), _w(p1, "ln1_g"), _w(p1, "ln1_b"),
        p1["W1"], _w(p1, "b1"), p1["W2"], _w(p1, "b2"),
        _w(p1, "ln2_g"), _w(p1, "ln2_b"),
        r0w, r0b.reshape(1, -1), r1w, r1b.reshape(1, -1),
        r2w, r2b.reshape(1, -1))

    return out


# speedup vs baseline: 14.9871x; 1.0231x over previous
import functools

import jax
import jax.numpy as jnp
from jax import lax
from jax.experimental import pallas as pl
from jax.experimental.pallas import tpu as pltpu
from jax.experimental.pallas import tpu_sc as plsc

N = 10000
E = 320000
D = 128
H = 8
DH = 16

NC = 2
NS = 16
DHALF = D // NC
HHALF = H // NC
EPT = E // NS
EB = 80
NBATCH = EPT // EB
CHUNK = 50
NCHUNK = NBATCH // CHUNK
RPT = 624

_SCALE = 1.0 / (DH ** 0.5)


def _edge_body(q_hbm, kv_hbm, packed_hbm, z64_hbm, z8_hbm,
               aggu_out, s_out,
               packed_all, src_all, dst_all,
               kv0, q0, ev0, eb0, kv1, q1, ev1, eb1,
               aggu_sp, s_sp,
               gq0, gk0, gq1, gk1, sa0, ss0, sa1, ss1):
    c = lax.axis_index("c")
    s = lax.axis_index("s")

    row0 = s * RPT
    pltpu.sync_copy(z64_hbm.at[pl.ds(0, RPT)], aggu_sp.at[pl.ds(row0, RPT)])
    pltpu.sync_copy(z8_hbm.at[pl.ds(0, RPT)], s_sp.at[pl.ds(row0, RPT)])
    pltpu.sync_copy(z8_hbm.at[pl.ds(0, EB)], eb0)
    pltpu.sync_copy(z8_hbm.at[pl.ds(0, EB)], eb1)

    @pl.when(s == NS - 1)
    def _tail_zero():
        pltpu.sync_copy(z64_hbm.at[pl.ds(0, N - NS * RPT)],
                        aggu_sp.at[pl.ds(NS * RPT, N - NS * RPT)])
        pltpu.sync_copy(z8_hbm.at[pl.ds(0, N - NS * RPT)],
                        s_sp.at[pl.ds(NS * RPT, N - NS * RPT)])

    plsc.subcore_barrier()

    lanes = lax.iota(jnp.int32, 16)

    def _compute(kvrows, qrows, evbuf, ebuf):
        def _group(g, gcarry):
            eids = g * 16 + lanes
            for h in range(HHALF):
                prods = []
                for f in range(DH):
                    kcol = jnp.full((16,), h * DH + f, jnp.int32)
                    qcol = kcol + c * DHALF
                    kt = plsc.load_gather(kvrows, [eids, kcol])
                    qt = plsc.load_gather(qrows, [eids, qcol])
                    prods.append(kt * qt)
                while len(prods) > 1:
                    prods = [prods[i] + prods[i + 1]
                             for i in range(0, len(prods), 2)]
                ev = jnp.exp(prods[0] * _SCALE)
                ecol = jnp.full((16,), h, jnp.int32) + c * HHALF
                plsc.store_scatter(ebuf, [eids, ecol], ev)
                for f in range(DH):
                    col = jnp.full((16,), h * DH + f, jnp.int32)
                    vt = plsc.load_gather(kvrows, [eids, col + DHALF])
                    plsc.store_scatter(evbuf, [eids, col], vt * ev)
            return gcarry
        lax.fori_loop(0, EB // 16, _group, 0)

    def _issue_gathers(b, kvrows, qrows, gq, gk):
        pltpu.async_copy(q_hbm.at[dst_all.at[b]], qrows, gq)
        pltpu.async_copy(kv_hbm.at[c].at[src_all.at[b]], kvrows, gk)

    def _wait_gathers(b, kvrows, qrows, gq, gk):
        pltpu.make_async_copy(q_hbm.at[dst_all.at[b]], qrows, gq).wait()
        pltpu.make_async_copy(kv_hbm.at[c].at[src_all.at[b]], kvrows, gk).wait()

    def _wait_scatters(b, evbuf, ebuf, sa, ss):
        pltpu.make_async_copy(evbuf, aggu_sp.at[dst_all.at[b]], sa).wait()
        pltpu.make_async_copy(ebuf, s_sp.at[dst_all.at[b]], ss).wait()

    def _chunk(ci, ccarry):
        pltpu.sync_copy(packed_hbm.at[s, ci], packed_all)

        def _unpack(b, carry):
            for j in range(EB // 16):
                p = packed_all[b, pl.ds(j * 16, 16)]
                src_all[b, pl.ds(j * 16, 16)] = lax.shift_right_logical(p, 14)
                dst_all[b, pl.ds(j * 16, 16)] = lax.bitwise_and(p, 16383)
            return carry
        lax.fori_loop(0, CHUNK, _unpack, 0)

        _issue_gathers(0, kv0, q0, gq0, gk0)

        def _iter(i, carry):
            b0 = 2 * i
            b1 = b0 + 1
            _issue_gathers(b1, kv1, q1, gq1, gk1)
            _wait_gathers(b0, kv0, q0, gq0, gk0)

            @pl.when(i > 0)
            def _():
                _wait_scatters(b0, ev0, eb0, sa0, ss0)
            _compute(kv0, q0, ev0, eb0)
            pltpu.async_copy(ev0, aggu_sp.at[dst_all.at[b0]], sa0, add=True)
            pltpu.async_copy(eb0, s_sp.at[dst_all.at[b0]], ss0, add=True)

            @pl.when(i < CHUNK // 2 - 1)
            def _():
                _issue_gathers(b0 + 2, kv0, q0, gq0, gk0)
            _wait_gathers(b1, kv1, q1, gq1, gk1)

            @pl.when(i > 0)
            def _():
                _wait_scatters(b1, ev1, eb1, sa1, ss1)
            _compute(kv1, q1, ev1, eb1)
            pltpu.async_copy(ev1, aggu_sp.at[dst_all.at[b1]], sa1, add=True)
            pltpu.async_copy(eb1, s_sp.at[dst_all.at[b1]], ss1, add=True)
            return carry
        lax.fori_loop(0, CHUNK // 2, _iter, 0)
        _wait_scatters(0, ev0, eb0, sa0, ss0)
        _wait_scatters(0, ev1, eb1, sa1, ss1)
        return ccarry
    lax.fori_loop(0, NCHUNK, _chunk, 0)

    plsc.subcore_barrier()
    pltpu.sync_copy(aggu_sp.at[pl.ds(row0, RPT)],
                    aggu_out.at[c, pl.ds(row0, RPT)])
    pltpu.sync_copy(s_sp.at[pl.ds(row0, RPT)],
                    s_out.at[c, pl.ds(row0, RPT)])

    @pl.when(s == NS - 1)
    def _tail_out():
        pltpu.sync_copy(aggu_sp.at[pl.ds(NS * RPT, N - NS * RPT)],
                        aggu_out.at[c, pl.ds(NS * RPT, N - NS * RPT)])
        pltpu.sync_copy(s_sp.at[pl.ds(NS * RPT, N - NS * RPT)],
                        s_out.at[c, pl.ds(NS * RPT, N - NS * RPT)])


_edge_kernel = functools.partial(
    pl.kernel,
    out_type=[
        jax.ShapeDtypeStruct((NC, N, DHALF), jnp.float32),
        jax.ShapeDtypeStruct((NC, N, H), jnp.float32),
    ],
    mesh=plsc.VectorSubcoreMesh(core_axis_name="c", subcore_axis_name="s"),
    compiler_params=pltpu.CompilerParams(
        needs_layout_passes=False, use_tc_tiling_on_sc=False),
    scratch_types=(
        [
            pltpu.VMEM((CHUNK, EB), jnp.int32),
            pltpu.VMEM((CHUNK, EB), jnp.int32),
            pltpu.VMEM((CHUNK, EB), jnp.int32),
        ]
        + [
            pltpu.VMEM((EB, D), jnp.float32),
            pltpu.VMEM((EB, D), jnp.float32),
            pltpu.VMEM((EB, DHALF), jnp.float32),
            pltpu.VMEM((EB, H), jnp.float32),
        ] * 2
        + [
            pltpu.VMEM_SHARED((N, DHALF), jnp.float32),
            pltpu.VMEM_SHARED((N, H), jnp.float32),
        ]
        + [pltpu.SemaphoreType.DMA] * 8
    ),
)(_edge_body)


BN = 400


def _ln(h, g, b):
    mu = jnp.mean(h, axis=-1, keepdims=True)
    var = jnp.mean((h - mu) ** 2, axis=-1, keepdims=True)
    return (h - mu) * lax.rsqrt(var + 1e-5) * g + b


def _dot(a, b):
    return jnp.dot(a, b, preferred_element_type=jnp.float32)


def _post_attn(ap, sp, h_in, wo, bo, g1, b1, w1, bf1, w2, bf2, g2, b2):
    aggu = jnp.concatenate([ap[0], ap[1]], axis=-1)
    s8 = sp[0] + sp[1]
    rows = lax.broadcasted_iota(jnp.int32, (H, D), 0)
    cols = lax.broadcasted_iota(jnp.int32, (H, D), 1)
    expand = (cols // DH == rows).astype(jnp.float32)
    denom = _dot(s8, expand) + 1e-9
    agg = aggu / denom
    attn = _dot(agg, wo) + bo + h_in
    h1 = _ln(attn, g1, b1)
    ff = _dot(jax.nn.relu(_dot(h1, w1) + bf1), w2) + bf2
    return _ln(h1 + ff, g2, b2)


def _split_qkv(h2, wq, bq, wk, bk, wv, bv, q_out, kv_out):
    q_out[...] = _dot(h2, wq) + bq
    k = _dot(h2, wk) + bk
    v = _dot(h2, wv) + bv
    kv_out[0] = jnp.concatenate([k[:, :DHALF], v[:, :DHALF]], axis=1)
    kv_out[1] = jnp.concatenate([k[:, DHALF:], v[:, DHALF:]], axis=1)


def _qkv_body(x_ref, wq, bq, wk, bk, wv, bv, q_out, kv_out):
    _split_qkv(x_ref[...], wq[...], bq[...], wk[...], bk[...], wv[...], bv[...],
               q_out, kv_out)


def _mid_body(h_ref, ap_ref, sp_ref,
              wo, bo, g1, b1, w1, bf1, w2, bf2, g2, b2,
              wq, bq, wk, bk, wv, bv,
              h_out, q_out, kv_out):
    h2 = _post_attn(ap_ref, sp_ref, h_ref[...],
                    wo[...], bo[...], g1[...], b1[...], w1[...], bf1[...],
                    w2[...], bf2[...], g2[...], b2[...])
    h_out[...] = h2
    _split_qkv(h2, wq[...], bq[...], wk[...], bk[...], wv[...], bv[...],
               q_out, kv_out)


def _final_body(h_ref, ap_ref, sp_ref,
                wo, bo, g1, b1, w1, bf1, w2, bf2, g2, b2,
                r0w, r0b, r1w, r1b, r2w, r2b,
                out_ref):
    h2 = _post_attn(ap_ref, sp_ref, h_ref[...],
                    wo[...], bo[...], g1[...], b1[...], w1[...], bf1[...],
                    w2[...], bf2[...], g2[...], b2[...])
    hc = jnp.concatenate([h_ref[...], h2], axis=1)
    r = jax.nn.relu(_dot(hc, r0w[...]) + r0b[...])
    r = jax.nn.relu(_dot(r, r1w[...]) + r1b[...])
    out_ref[...] = _dot(r, r2w[...]) + r2b[...]


def _rowspec(cols):
    return pl.BlockSpec((BN, cols), lambda i: (i, 0))


def _fullspec(shape):
    nd = len(shape)
    return pl.BlockSpec(shape, lambda i, _nd=nd: (0,) * _nd)


def _partspec(cols):
    return pl.BlockSpec((NC, BN, cols), lambda i: (0, i, 0))


def _w(p, name):
    arr = p[name]
    if arr.ndim == 1:
        arr = arr.reshape(1, -1)
    return arr


def kernel(x, edge_index, params):
    packed = (jnp.left_shift(edge_index[0], 14) | edge_index[1]).reshape(
        NS, NCHUNK, CHUNK, EB)
    z64 = jnp.zeros((N, DHALF), jnp.float32)
    z8 = jnp.zeros((N, H), jnp.float32)
    p0, p1 = params["layers"]
    (r0w, r0b), (r1w, r1b), (r2w, r2b) = params["readout"]

    grid = (N // BN,)

    qkv_call = pl.pallas_call(
        _qkv_body,
        grid=grid,
        in_specs=[_rowspec(D)] + [_fullspec(s) for s in
                  [(D, D), (1, D), (D, D), (1, D), (D, D), (1, D)]],
        out_specs=[_rowspec(D), _partspec(D)],
        out_shape=[jax.ShapeDtypeStruct((N, D), jnp.float32),
                   jax.ShapeDtypeStruct((NC, N, D), jnp.float32)],
    )

    q0, kv0 = qkv_call(
        x, p0["Wq"], _w(p0, "bq"), p0["Wk"], _w(p0, "bk"),
        p0["Wv"], _w(p0, "bv"))

    ap0, sp0 = _edge_kernel(q0, kv0, packed, z64, z8)

    post_w_shapes = [(D, D), (1, D), (1, D), (1, D), (D, 2 * D), (1, 2 * D),
                     (2 * D, D), (1, D), (1, D), (1, D)]
    qkv_w_shapes = [(D, D), (1, D)] * 3

    mid_call = pl.pallas_call(
        _mid_body,
        grid=grid,
        in_specs=[_rowspec(D), _partspec(DHALF), _partspec(H)]
                 + [_fullspec(s) for s in post_w_shapes + qkv_w_shapes],
        out_specs=[_rowspec(D), _rowspec(D), _partspec(D)],
        out_shape=[jax.ShapeDtypeStruct((N, D), jnp.float32),
                   jax.ShapeDtypeStruct((N, D), jnp.float32),
                   jax.ShapeDtypeStruct((NC, N, D), jnp.float32)],
    )

    h1, q1, kv1 = mid_call(
        x, ap0, sp0,
        p0["Wo"], _w(p0, "bo"), _w(p0, "ln1_g"), _w(p0, "ln1_b"),
        p0["W1"], _w(p0, "b1"), p0["W2"], _w(p0, "b2"),
        _w(p0, "ln2_g"), _w(p0, "ln2_b"),
        p1["Wq"], _w(p1, "bq"), p1["Wk"], _w(p1, "bk"),
        p1["Wv"], _w(p1, "bv"))

    ap1, sp1 = _edge_kernel(q1, kv1, packed, z64, z8)

    readout_shapes = [(2 * D, D), (1, D), (D, D // 2), (1, D // 2),
                      (D // 2, 10), (1, 10)]

    final_call = pl.pallas_call(
        _final_body,
        grid=grid,
        in_specs=[_rowspec(D), _partspec(DHALF), _partspec(H)]
                 + [_fullspec(s) for s in post_w_shapes + readout_shapes],
        out_specs=_rowspec(10),
        out_shape=jax.ShapeDtypeStruct((N, 10), jnp.float32),
    )

    out = final_call(
        h1, ap1, sp1,
        p1["Wo"], _w(p1, "bo"), _w(p1, "ln1_g"), _w(p1, "ln1_b"),
        p1["W1"], _w(p1, "b1"), p1["W2"], _w(p1, "b2"),
        _w(p1, "ln2_g"), _w(p1, "ln2_b"),
        r0w, r0b.reshape(1, -1), r1w, r1b.reshape(1, -1),
        r2w, r2b.reshape(1, -1))

    return out


# speedup vs baseline: 16.5132x; 1.1018x over previous
import functools

import jax
import jax.numpy as jnp
from jax import lax
from jax.experimental import pallas as pl
from jax.experimental.pallas import tpu as pltpu
from jax.experimental.pallas import tpu_sc as plsc

N = 10000
E = 320000
D = 128
H = 8
DH = 16

NC = 2
NS = 16
DHALF = D // NC
HHALF = H // NC
EPT = E // NS
EB = 80
NBATCH = EPT // EB
CHUNK = 50
NCHUNK = NBATCH // CHUNK
RPT = 624

_SCALE = 1.0 / (DH ** 0.5)


def _edge_body(q_hbm, kv_hbm, packed_hbm, z64_hbm, z8_hbm,
               aggu_out, s_out,
               packed_all, src_all, dst_all,
               kv0, q0, ev0, eb0, kv1, q1, ev1, eb1,
               aggu_sp, s_sp,
               gq0, gk0, gq1, gk1, sa0, ss0, sa1, ss1):
    c = lax.axis_index("c")
    s = lax.axis_index("s")

    row0 = s * RPT
    pltpu.sync_copy(z64_hbm.at[pl.ds(0, RPT)], aggu_sp.at[pl.ds(row0, RPT)])
    pltpu.sync_copy(z8_hbm.at[pl.ds(0, RPT)], s_sp.at[pl.ds(row0, RPT)])
    pltpu.sync_copy(z8_hbm.at[pl.ds(0, EB)], eb0)
    pltpu.sync_copy(z8_hbm.at[pl.ds(0, EB)], eb1)

    @pl.when(s == NS - 1)
    def _tail_zero():
        pltpu.sync_copy(z64_hbm.at[pl.ds(0, N - NS * RPT)],
                        aggu_sp.at[pl.ds(NS * RPT, N - NS * RPT)])
        pltpu.sync_copy(z8_hbm.at[pl.ds(0, N - NS * RPT)],
                        s_sp.at[pl.ds(NS * RPT, N - NS * RPT)])

    plsc.subcore_barrier()

    lanes = lax.iota(jnp.int32, 16)

    def _compute(kvrows, qrows, evbuf, ebuf):
        def _edge(i, carry):
            for u in range(2):
                e = 2 * i + u
                er = jnp.zeros((16,), jnp.float32)
                for h in range(HHALF):
                    k = kvrows[e, pl.ds(h * DH, DH)]
                    q = qrows[e, pl.ds(c * DHALF + h * DH, DH)]
                    sc = jnp.sum(k * q, axis=0) * _SCALE
                    evb = jnp.exp(jnp.broadcast_to(sc, (16,)))
                    v = kvrows[e, pl.ds(DHALF + h * DH, DH)]
                    evbuf[e, pl.ds(h * DH, DH)] = v * evb
                    er = jnp.where(lanes == c * HHALF + h, evb, er)
                ebuf[e, :] = er
            return carry
        lax.fori_loop(0, EB // 2, _edge, 0)

    def _issue_gathers(b, kvrows, qrows, gq, gk):
        pltpu.async_copy(q_hbm.at[dst_all.at[b]], qrows, gq)
        pltpu.async_copy(kv_hbm.at[c].at[src_all.at[b]], kvrows, gk)

    def _wait_gathers(b, kvrows, qrows, gq, gk):
        pltpu.make_async_copy(q_hbm.at[dst_all.at[b]], qrows, gq).wait()
        pltpu.make_async_copy(kv_hbm.at[c].at[src_all.at[b]], kvrows, gk).wait()

    def _wait_scatters(b, evbuf, ebuf, sa, ss):
        pltpu.make_async_copy(evbuf, aggu_sp.at[dst_all.at[b]], sa).wait()
        pltpu.make_async_copy(ebuf, s_sp.at[dst_all.at[b]], ss).wait()

    def _chunk(ci, ccarry):
        pltpu.sync_copy(packed_hbm.at[s, ci], packed_all)

        def _unpack(b, carry):
            for j in range(EB // 16):
                p = packed_all[b, pl.ds(j * 16, 16)]
                src_all[b, pl.ds(j * 16, 16)] = lax.shift_right_logical(p, 14)
                dst_all[b, pl.ds(j * 16, 16)] = lax.bitwise_and(p, 16383)
            return carry
        lax.fori_loop(0, CHUNK, _unpack, 0)

        _issue_gathers(0, kv0, q0, gq0, gk0)

        def _iter(i, carry):
            b0 = 2 * i
            b1 = b0 + 1
            _issue_gathers(b1, kv1, q1, gq1, gk1)
            _wait_gathers(b0, kv0, q0, gq0, gk0)

            @pl.when(i > 0)
            def _():
                _wait_scatters(b0, ev0, eb0, sa0, ss0)
            _compute(kv0, q0, ev0, eb0)
            pltpu.async_copy(ev0, aggu_sp.at[dst_all.at[b0]], sa0, add=True)
            pltpu.async_copy(eb0, s_sp.at[dst_all.at[b0]], ss0, add=True)

            @pl.when(i < CHUNK // 2 - 1)
            def _():
                _issue_gathers(b0 + 2, kv0, q0, gq0, gk0)
            _wait_gathers(b1, kv1, q1, gq1, gk1)

            @pl.when(i > 0)
            def _():
                _wait_scatters(b1, ev1, eb1, sa1, ss1)
            _compute(kv1, q1, ev1, eb1)
            pltpu.async_copy(ev1, aggu_sp.at[dst_all.at[b1]], sa1, add=True)
            pltpu.async_copy(eb1, s_sp.at[dst_all.at[b1]], ss1, add=True)
            return carry
        lax.fori_loop(0, CHUNK // 2, _iter, 0)
        _wait_scatters(0, ev0, eb0, sa0, ss0)
        _wait_scatters(0, ev1, eb1, sa1, ss1)
        return ccarry
    lax.fori_loop(0, NCHUNK, _chunk, 0)

    plsc.subcore_barrier()
    pltpu.sync_copy(aggu_sp.at[pl.ds(row0, RPT)],
                    aggu_out.at[c, pl.ds(row0, RPT)])
    pltpu.sync_copy(s_sp.at[pl.ds(row0, RPT)],
                    s_out.at[c, pl.ds(row0, RPT)])

    @pl.when(s == NS - 1)
    def _tail_out():
        pltpu.sync_copy(aggu_sp.at[pl.ds(NS * RPT, N - NS * RPT)],
                        aggu_out.at[c, pl.ds(NS * RPT, N - NS * RPT)])
        pltpu.sync_copy(s_sp.at[pl.ds(NS * RPT, N - NS * RPT)],
                        s_out.at[c, pl.ds(NS * RPT, N - NS * RPT)])


_edge_kernel = functools.partial(
    pl.kernel,
    out_type=[
        jax.ShapeDtypeStruct((NC, N, DHALF), jnp.float32),
        jax.ShapeDtypeStruct((NC, N, 16), jnp.float32),
    ],
    mesh=plsc.VectorSubcoreMesh(core_axis_name="c", subcore_axis_name="s"),
    compiler_params=pltpu.CompilerParams(
        needs_layout_passes=False, use_tc_tiling_on_sc=False),
    scratch_types=(
        [
            pltpu.VMEM((CHUNK, EB), jnp.int32),
            pltpu.VMEM((CHUNK, EB), jnp.int32),
            pltpu.VMEM((CHUNK, EB), jnp.int32),
        ]
        + [
            pltpu.VMEM((EB, D), jnp.float32),
            pltpu.VMEM((EB, D), jnp.float32),
            pltpu.VMEM((EB, DHALF), jnp.float32),
            pltpu.VMEM((EB, 16), jnp.float32),
        ] * 2
        + [
            pltpu.VMEM_SHARED((N, DHALF), jnp.float32),
            pltpu.VMEM_SHARED((N, 16), jnp.float32),
        ]
        + [pltpu.SemaphoreType.DMA] * 8
    ),
)(_edge_body)


BN = 400


def _ln(h, g, b):
    mu = jnp.mean(h, axis=-1, keepdims=True)
    var = jnp.mean((h - mu) ** 2, axis=-1, keepdims=True)
    return (h - mu) * lax.rsqrt(var + 1e-5) * g + b


def _dot(a, b):
    return jnp.dot(a, b, preferred_element_type=jnp.float32)


def _post_attn(ap, sp, h_in, wo, bo, g1, b1, w1, bf1, w2, bf2, g2, b2):
    aggu = jnp.concatenate([ap[0], ap[1]], axis=-1)
    s8 = sp[0] + sp[1]
    rows = lax.broadcasted_iota(jnp.int32, (16, D), 0)
    cols = lax.broadcasted_iota(jnp.int32, (16, D), 1)
    expand = (cols // DH == rows).astype(jnp.float32)
    denom = _dot(s8, expand) + 1e-9
    agg = aggu / denom
    attn = _dot(agg, wo) + bo + h_in
    h1 = _ln(attn, g1, b1)
    ff = _dot(jax.nn.relu(_dot(h1, w1) + bf1), w2) + bf2
    return _ln(h1 + ff, g2, b2)


def _split_qkv(h2, wq, bq, wk, bk, wv, bv, q_out, kv_out):
    q_out[...] = _dot(h2, wq) + bq
    k = _dot(h2, wk) + bk
    v = _dot(h2, wv) + bv
    kv_out[0] = jnp.concatenate([k[:, :DHALF], v[:, :DHALF]], axis=1)
    kv_out[1] = jnp.concatenate([k[:, DHALF:], v[:, DHALF:]], axis=1)


def _qkv_body(x_ref, wq, bq, wk, bk, wv, bv, q_out, kv_out):
    _split_qkv(x_ref[...], wq[...], bq[...], wk[...], bk[...], wv[...], bv[...],
               q_out, kv_out)


def _mid_body(h_ref, ap_ref, sp_ref,
              wo, bo, g1, b1, w1, bf1, w2, bf2, g2, b2,
              wq, bq, wk, bk, wv, bv,
              h_out, q_out, kv_out):
    h2 = _post_attn(ap_ref, sp_ref, h_ref[...],
                    wo[...], bo[...], g1[...], b1[...], w1[...], bf1[...],
                    w2[...], bf2[...], g2[...], b2[...])
    h_out[...] = h2
    _split_qkv(h2, wq[...], bq[...], wk[...], bk[...], wv[...], bv[...],
               q_out, kv_out)


def _final_body(h_ref, ap_ref, sp_ref,
                wo, bo, g1, b1, w1, bf1, w2, bf2, g2, b2,
                r0w, r0b, r1w, r1b, r2w, r2b,
                out_ref):
    h2 = _post_attn(ap_ref, sp_ref, h_ref[...],
                    wo[...], bo[...], g1[...], b1[...], w1[...], bf1[...],
                    w2[...], bf2[...], g2[...], b2[...])
    hc = jnp.concatenate([h_ref[...], h2], axis=1)
    r = jax.nn.relu(_dot(hc, r0w[...]) + r0b[...])
    r = jax.nn.relu(_dot(r, r1w[...]) + r1b[...])
    out_ref[...] = _dot(r, r2w[...]) + r2b[...]


def _rowspec(cols):
    return pl.BlockSpec((BN, cols), lambda i: (i, 0))


def _fullspec(shape):
    nd = len(shape)
    return pl.BlockSpec(shape, lambda i, _nd=nd: (0,) * _nd)


def _partspec(cols):
    return pl.BlockSpec((NC, BN, cols), lambda i: (0, i, 0))


def _w(p, name):
    arr = p[name]
    if arr.ndim == 1:
        arr = arr.reshape(1, -1)
    return arr


def kernel(x, edge_index, params):
    packed = (jnp.left_shift(edge_index[0], 14) | edge_index[1]).reshape(
        NS, NCHUNK, CHUNK, EB)
    z64 = jnp.zeros((N, DHALF), jnp.float32)
    z16 = jnp.zeros((N, 16), jnp.float32)
    p0, p1 = params["layers"]
    (r0w, r0b), (r1w, r1b), (r2w, r2b) = params["readout"]

    grid = (N // BN,)

    qkv_call = pl.pallas_call(
        _qkv_body,
        grid=grid,
        in_specs=[_rowspec(D)] + [_fullspec(s) for s in
                  [(D, D), (1, D), (D, D), (1, D), (D, D), (1, D)]],
        out_specs=[_rowspec(D), _partspec(D)],
        out_shape=[jax.ShapeDtypeStruct((N, D), jnp.float32),
                   jax.ShapeDtypeStruct((NC, N, D), jnp.float32)],
    )

    q0, kv0 = qkv_call(
        x, p0["Wq"], _w(p0, "bq"), p0["Wk"], _w(p0, "bk"),
        p0["Wv"], _w(p0, "bv"))

    ap0, sp0 = _edge_kernel(q0, kv0, packed, z64, z16)

    post_w_shapes = [(D, D), (1, D), (1, D), (1, D), (D, 2 * D), (1, 2 * D),
                     (2 * D, D), (1, D), (1, D), (1, D)]
    qkv_w_shapes = [(D, D), (1, D)] * 3

    mid_call = pl.pallas_call(
        _mid_body,
        grid=grid,
        in_specs=[_rowspec(D), _partspec(DHALF), _partspec(16)]
                 + [_fullspec(s) for s in post_w_shapes + qkv_w_shapes],
        out_specs=[_rowspec(D), _rowspec(D), _partspec(D)],
        out_shape=[jax.ShapeDtypeStruct((N, D), jnp.float32),
                   jax.ShapeDtypeStruct((N, D), jnp.float32),
                   jax.ShapeDtypeStruct((NC, N, D), jnp.float32)],
    )

    h1, q1, kv1 = mid_call(
        x, ap0, sp0,
        p0["Wo"], _w(p0, "bo"), _w(p0, "ln1_g"), _w(p0, "ln1_b"),
        p0["W1"], _w(p0, "b1"), p0["W2"], _w(p0, "b2"),
        _w(p0, "ln2_g"), _w(p0, "ln2_b"),
        p1["Wq"], _w(p1, "bq"), p1["Wk"], _w(p1, "bk"),
        p1["Wv"], _w(p1, "bv"))

    ap1, sp1 = _edge_kernel(q1, kv1, packed, z64, z16)

    readout_shapes = [(2 * D, D), (1, D), (D, D // 2), (1, D // 2),
                      (D // 2, 10), (1, 10)]

    final_call = pl.pallas_call(
        _final_body,
        grid=grid,
        in_specs=[_rowspec(D), _partspec(DHALF), _partspec(16)]
                 + [_fullspec(s) for s in post_w_shapes + readout_shapes],
        out_specs=_rowspec(10),
        out_shape=jax.ShapeDtypeStruct((N, 10), jnp.float32),
    )

    out = final_call(
        h1, ap1, sp1,
        p1["Wo"], _w(p1, "bo"), _w(p1, "ln1_g"), _w(p1, "ln1_b"),
        p1["W1"], _w(p1, "b1"), p1["W2"], _w(p1, "b2"),
        _w(p1, "ln2_g"), _w(p1, "ln2_b"),
        r0w, r0b.reshape(1, -1), r1w, r1b.reshape(1, -1),
        r2w, r2b.reshape(1, -1))

    return out
